# baseline (speedup 1.0000x reference)
import functools
import jax
import jax.numpy as jnp
from jax import lax
from jax.experimental import pallas as pl
from jax.experimental.pallas import tpu as pltpu
from jax.experimental.pallas import tpu_sc as plsc

B, N, C = 2, 2048, 256
H = 8
G = C // H
K = 16
BN = B * N
E = BN * K
EPS = 1e-5
NW = 32
CH = 128
PER_W = E // NW
TW = 384

_NEG = float('-inf')


def _leaky(x):
    return jnp.where(x >= 0, x, 0.2 * x)


def _proj1_body(xf_ref, wq_ref, wk_ref, wv_ref, wp_ref, y_ref, st_ref):
    i = pl.program_id(0)
    xb = xf_ref[...]

    def mm(w_ref):
        return lax.dot_general(xb, w_ref[...], (((1,), (1,)), ((), ())),
                               preferred_element_type=jnp.float32)

    y = jnp.concatenate([mm(wq_ref), mm(wk_ref), mm(wv_ref), mm(wp_ref)],
                        axis=1)
    y_ref[...] = y
    s = jnp.sum(y, axis=0, keepdims=True)
    s2 = jnp.sum(y * y, axis=0, keepdims=True)
    part = jnp.concatenate(
        [jnp.broadcast_to(s, (8, 4 * C)), jnp.broadcast_to(s2, (8, 4 * C))], axis=0)

    @pl.when(i == 0)
    def _():
        st_ref[...] = jnp.zeros_like(st_ref)

    st_ref[...] += part


def _proj2_body(y_ref, st_ref, gb_ref, we1_ref,
                kn_ref, vn_ref, tab_ref, qe1_ref):
    cnt = jnp.float32(BN)
    mean = st_ref[0:1, :] / cnt
    ex2 = st_ref[8:9, :] / cnt
    var = ex2 - mean * mean
    inv = lax.rsqrt(var + EPS)
    y = y_ref[...]
    act = _leaky((y - mean) * inv * gb_ref[0:1, :] + gb_ref[1:2, :])
    qn = act[:, 0:C]
    kn = act[:, C:2 * C]
    vn = act[:, 2 * C:3 * C]
    kn_ref[...] = kn
    vn_ref[...] = vn
    tab_ref[:, 0:C] = y[:, 3 * C:4 * C]
    we1 = we1_ref[...]

    def mm(a):
        return lax.dot_general(a, we1, (((1,), (1,)), ((), ())),
                               preferred_element_type=jnp.float32)

    qe1_ref[...] = mm(qn)
    tab_ref[:, C:C + G] = mm(kn)
    tab_ref[:, C + G:TW] = jnp.zeros((kn.shape[0], TW - C - G), jnp.float32)


def _topk_body(xb_ref, xa_ref, idx_ref):
    b = pl.program_id(0)
    xb = xb_ref[0]
    xa = xa_ref[0]
    d = lax.dot_general(xb, xa, (((1,), (1,)), ((), ())),
                        preferred_element_type=jnp.float32)
    sqb = jnp.sum(xb * xb, axis=1, keepdims=True)
    sqa = jnp.sum(xa * xa, axis=1)
    nd = 2.0 * d - sqb - sqa[None, :]
    r = nd.shape[0]
    iota = lax.broadcasted_iota(jnp.int32, (r, N), 1)
    cols = []
    for _ in range(K):
        m = jnp.max(nd, axis=1, keepdims=True)
        sel = jnp.where(nd >= m, iota, N)
        a = jnp.min(sel, axis=1, keepdims=True)
        cols.append(a)
        nd = jnp.where(iota == a, _NEG, nd)
    idx = jnp.concatenate(cols, axis=1)
    idx_ref[...] = idx + b * N


def _sc_gather_body(idx_hbm, tab_hbm, out_hbm, idx_v, buf_a, buf_b,
                    sem_a, sem_b):
    wid = lax.axis_index("s") * 2 + lax.axis_index("c")
    wbase = pl.multiple_of(wid * PER_W, CH)
    nch = PER_W // CH

    pltpu.sync_copy(idx_hbm.at[pl.ds(pl.multiple_of(wid * nch, 8), nch)], idx_v)
    pltpu.async_copy(tab_hbm.at[idx_v.at[0]], buf_a, sem_a)

    def body(j, carry):
        i0 = 2 * j
        pltpu.async_copy(tab_hbm.at[idx_v.at[i0 + 1]], buf_b, sem_b)
        pltpu.make_async_copy(tab_hbm.at[idx_v.at[i0]], buf_a, sem_a).wait()
        pltpu.sync_copy(buf_a, out_hbm.at[pl.ds(wbase + i0 * CH, CH)])

        @pl.when(i0 + 2 < nch)
        def _():
            pltpu.async_copy(tab_hbm.at[idx_v.at[i0 + 2]], buf_a, sem_a)

        pltpu.make_async_copy(tab_hbm.at[idx_v.at[i0 + 1]], buf_b, sem_b).wait()
        pltpu.sync_copy(buf_b, out_hbm.at[pl.ds(wbase + (i0 + 1) * CH, CH)])
        return carry

    lax.fori_loop(0, nch // 2, body, 0)


def _sc_gatherk_body(idx_hbm, tab_hbm, ppe_hbm, ke1e_hbm, idx_v, buf_a, buf_b,
                     sem_a, sem_b):
    wid = lax.axis_index("s") * 2 + lax.axis_index("c")
    wbase = pl.multiple_of(wid * PER_W, CH)
    nch = PER_W // CH

    pltpu.sync_copy(idx_hbm.at[pl.ds(pl.multiple_of(wid * nch, 8), nch)], idx_v)
    pltpu.async_copy(tab_hbm.at[idx_v.at[0]], buf_a, sem_a)

    def wb(buf, i):
        pltpu.sync_copy(buf.at[:, pl.ds(0, C)],
                        ppe_hbm.at[pl.ds(wbase + i * CH, CH)])
        pltpu.sync_copy(buf.at[:, pl.ds(C, 128)],
                        ke1e_hbm.at[pl.ds(wbase + i * CH, CH)])

    def body(j, carry):
        i0 = 2 * j
        pltpu.async_copy(tab_hbm.at[idx_v.at[i0 + 1]], buf_b, sem_b)
        pltpu.make_async_copy(tab_hbm.at[idx_v.at[i0]], buf_a, sem_a).wait()
        wb(buf_a, i0)

        @pl.when(i0 + 2 < nch)
        def _():
            pltpu.async_copy(tab_hbm.at[idx_v.at[i0 + 2]], buf_a, sem_a)

        pltpu.make_async_copy(tab_hbm.at[idx_v.at[i0 + 1]], buf_b, sem_b).wait()
        wb(buf_b, i0 + 1)
        return carry

    lax.fori_loop(0, nch // 2, body, 0)


def _pbstat_body(ppe_ref, pp_ref, st_ref):
    i = pl.program_id(0)
    y = ppe_ref[...] - pp_ref[...][:, None, :]
    s = jnp.sum(y, axis=(0, 1))[None, :]
    s2 = jnp.sum(y * y, axis=(0, 1))[None, :]
    part = jnp.concatenate(
        [jnp.broadcast_to(s, (8, C)), jnp.broadcast_to(s2, (8, C))], axis=0)

    @pl.when(i == 0)
    def _():
        st_ref[...] = jnp.zeros_like(st_ref)

    st_ref[...] += part


def _e1_body(ppe_ref, pp_ref, stpb_ref, gbpb_ref, qe1_ref, ke1e_ref, we1_ref,
             u1_ref, st_ref):
    i = pl.program_id(0)
    cnt = jnp.float32(E)
    mean = stpb_ref[0:1, :] / cnt
    ex2 = stpb_ref[8:9, :] / cnt
    inv = lax.rsqrt(ex2 - mean * mean + EPS)
    y = ppe_ref[...] - pp_ref[...][:, None, :]
    posb = _leaky((y - mean[None, :, :]) * inv[None, :, :]
                  * gbpb_ref[0:1, :][None, :, :] + gbpb_ref[1:2, :][None, :, :])
    r = posb.shape[0]
    u1 = lax.dot_general(posb.reshape(r * K, C), we1_ref[...],
                         (((1,), (1,)), ((), ())),
                         preferred_element_type=jnp.float32).reshape(r, K, G)
    u1 = u1 + ke1e_ref[:, :, 0:G] - qe1_ref[...][:, None, :]
    u1_ref[...] = u1
    s = jnp.sum(u1, axis=(0, 1))[None, :]
    s2 = jnp.sum(u1 * u1, axis=(0, 1))[None, :]
    part = jnp.concatenate(
        [jnp.broadcast_to(s, (8, G)), jnp.broadcast_to(s2, (8, G))], axis=0)

    @pl.when(i == 0)
    def _():
        st_ref[...] = jnp.zeros_like(st_ref)

    st_ref[...] += part


def _e2_body(u1_ref, st1_ref, gbe1_ref, we2_ref, u2_ref, st_ref):
    i = pl.program_id(0)
    cnt = jnp.float32(E)
    mean = st1_ref[0:1, :] / cnt
    ex2 = st1_ref[8:9, :] / cnt
    inv = lax.rsqrt(ex2 - mean * mean + EPS)
    t1 = _leaky((u1_ref[...] - mean[None, :, :]) * inv[None, :, :]
                * gbe1_ref[0:1, :][None, :, :] + gbe1_ref[1:2, :][None, :, :])
    r = t1.shape[0]
    u2 = lax.dot_general(t1.reshape(r * K, G), we2_ref[...],
                         (((1,), (1,)), ((), ())),
                         preferred_element_type=jnp.float32).reshape(r, K, G)
    u2_ref[...] = u2
    s = jnp.sum(u2, axis=(0, 1))[None, :]
    s2 = jnp.sum(u2 * u2, axis=(0, 1))[None, :]
    part = jnp.concatenate(
        [jnp.broadcast_to(s, (8, G)), jnp.broadcast_to(s2, (8, G))], axis=0)

    @pl.when(i == 0)
    def _():
        st_ref[...] = jnp.zeros_like(st_ref)

    st_ref[...] += part


def _final_body(u2_ref, st2_ref, gbe2_ref, ppe_ref, pp_ref, stpb_ref,
                gbpb_ref, ve_ref, erep_ref, out_ref):
    cnt = jnp.float32(E)
    mean2 = st2_ref[0:1, :] / cnt
    ex22 = st2_ref[8:9, :] / cnt
    inv2 = lax.rsqrt(ex22 - mean2 * mean2 + EPS)
    t2 = _leaky((u2_ref[...] - mean2[None, :, :]) * inv2[None, :, :]
                * gbe2_ref[0:1, :][None, :, :] + gbe2_ref[1:2, :][None, :, :])
    mx = jnp.max(t2, axis=1, keepdims=True)
    exv = jnp.exp(t2 - mx)
    attn = exv / jnp.sum(exv, axis=1, keepdims=True)
    r = attn.shape[0]
    attn_c = jnp.dot(attn.reshape(r * K, G), erep_ref[...],
                     preferred_element_type=jnp.float32).reshape(r, K, C)
    meanp = stpb_ref[0:1, :] / cnt
    ex2p = stpb_ref[8:9, :] / cnt
    invp = lax.rsqrt(ex2p - meanp * meanp + EPS)
    y = ppe_ref[...] - pp_ref[...][:, None, :]
    posb = _leaky((y - meanp[None, :, :]) * invp[None, :, :]
                  * gbpb_ref[0:1, :][None, :, :] + gbpb_ref[1:2, :][None, :, :])
    value = ve_ref[...] + posb
    out_ref[...] = jnp.sum(value * attn_c, axis=1)


def kernel(x, Wq, bq, gq, betaq, Wk, bk, gk, betak, Wv, bv, gv, betav,
           Wpb, bpb, gpb, betapb, We1, be1, ge1, betae1, We2, be2, ge2, betae2):
    f32 = jnp.float32
    xf = x.reshape(BN, C)

    gam = jnp.concatenate([gq, gk, gv, jnp.ones((C,), f32)])
    bet = jnp.concatenate([betaq, betak, betav, jnp.zeros((C,), f32)])
    gb_qkv = jnp.zeros((8, 4 * C), f32).at[0].set(gam).at[1].set(bet)
    gb_pb = jnp.zeros((8, C), f32).at[0].set(gpb).at[1].set(betapb)
    gb_e1 = jnp.zeros((8, G), f32).at[0].set(ge1).at[1].set(betae1)
    gb_e2 = jnp.zeros((8, G), f32).at[0].set(ge2).at[1].set(betae2)
    erep = jnp.repeat(jnp.eye(G, dtype=f32), H, axis=1).reshape(G, C)

    R1 = 512
    wspec = pl.BlockSpec((C, C), lambda i: (0, 0))
    y_all, st_qkv = pl.pallas_call(
        _proj1_body,
        grid=(BN // R1,),
        in_specs=[pl.BlockSpec((R1, C), lambda i: (i, 0)),
                  wspec, wspec, wspec, wspec],
        out_specs=[pl.BlockSpec((R1, 4 * C), lambda i: (i, 0)),
                   pl.BlockSpec((16, 4 * C), lambda i: (0, 0))],
        out_shape=[jax.ShapeDtypeStruct((BN, 4 * C), f32),
                   jax.ShapeDtypeStruct((16, 4 * C), f32)],
    )(xf, Wq, Wk, Wv, Wpb)

    R2 = 512
    kn, vn, tab, qe1 = pl.pallas_call(
        _proj2_body,
        grid=(BN // R2,),
        in_specs=[pl.BlockSpec((R2, 4 * C), lambda i: (i, 0)),
                  pl.BlockSpec((16, 4 * C), lambda i: (0, 0)),
                  pl.BlockSpec((8, 4 * C), lambda i: (0, 0)),
                  pl.BlockSpec((G, C), lambda i: (0, 0))],
        out_specs=[pl.BlockSpec((R2, C), lambda i: (i, 0)),
                   pl.BlockSpec((R2, C), lambda i: (i, 0)),
                   pl.BlockSpec((R2, TW), lambda i: (i, 0)),
                   pl.BlockSpec((R2, G), lambda i: (i, 0))],
        out_shape=[jax.ShapeDtypeStruct((BN, C), f32),
                   jax.ShapeDtypeStruct((BN, C), f32),
                   jax.ShapeDtypeStruct((BN, TW), f32),
                   jax.ShapeDtypeStruct((BN, G), f32)],
    )(y_all, st_qkv, gb_qkv, We1)

    R3 = 256

    def _run_topk(feat):
        return pl.pallas_call(
            _topk_body,
            grid=(B, N // R3),
            in_specs=[pl.BlockSpec((1, R3, C), lambda b, j: (b, j, 0)),
                      pl.BlockSpec((1, N, C), lambda b, j: (b, 0, 0))],
            out_specs=pl.BlockSpec((R3, K), lambda b, j: (b * (N // R3) + j, 0)),
            out_shape=jax.ShapeDtypeStruct((BN, K), jnp.int32),
        )(feat, feat)

    idxg = _run_topk(kn.reshape(B, N, C))
    idxgv = _run_topk(vn.reshape(B, N, C))

    mesh = plsc.VectorSubcoreMesh(core_axis_name="c", subcore_axis_name="s")

    def _scratch(width):
        return [pltpu.VMEM((E // NW // CH, CH), jnp.int32),
                pltpu.VMEM((CH, width), f32),
                pltpu.VMEM((CH, width), f32),
                pltpu.SemaphoreType.DMA,
                pltpu.SemaphoreType.DMA]

    gather_k = functools.partial(
        pl.kernel, mesh=mesh,
        out_type=[jax.ShapeDtypeStruct((E, C), f32),
                  jax.ShapeDtypeStruct((E, 128), f32)],
        scratch_types=_scratch(TW),
    )(_sc_gatherk_body)
    ppe_f, ke1e_f = gather_k(idxg.reshape(E // CH, CH), tab)

    gather_v = functools.partial(
        pl.kernel, mesh=mesh,
        out_type=jax.ShapeDtypeStruct((E, C), f32),
        scratch_types=_scratch(C),
    )(_sc_gather_body)
    ve_f = gather_v(idxgv.reshape(E // CH, CH), vn)

    ppe = ppe_f.reshape(BN, K, C)
    ke1e = ke1e_f.reshape(BN, K, 128)
    ve = ve_f.reshape(BN, K, C)

    R4 = 128
    st_pb = pl.pallas_call(
        _pbstat_body,
        grid=(BN // R4,),
        in_specs=[pl.BlockSpec((R4, K, C), lambda i: (i, 0, 0)),
                  pl.BlockSpec((R4, C), lambda i: (i, 0))],
        out_specs=pl.BlockSpec((16, C), lambda i: (0, 0)),
        out_shape=jax.ShapeDtypeStruct((16, C), f32),
    )(ppe, tab)

    u1e, st_u1 = pl.pallas_call(
        _e1_body,
        grid=(BN // R4,),
        in_specs=[pl.BlockSpec((R4, K, C), lambda i: (i, 0, 0)),
                  pl.BlockSpec((R4, C), lambda i: (i, 0)),
                  pl.BlockSpec((16, C), lambda i: (0, 0)),
                  pl.BlockSpec((8, C), lambda i: (0, 0)),
                  pl.BlockSpec((R4, G), lambda i: (i, 0)),
                  pl.BlockSpec((R4, K, 128), lambda i: (i, 0, 0)),
                  pl.BlockSpec((G, C), lambda i: (0, 0))],
        out_specs=[pl.BlockSpec((R4, K, G), lambda i: (i, 0, 0)),
                   pl.BlockSpec((16, G), lambda i: (0, 0))],
        out_shape=[jax.ShapeDtypeStruct((BN, K, G), f32),
                   jax.ShapeDtypeStruct((16, G), f32)],
    )(ppe, tab, st_pb, gb_pb, qe1, ke1e, We1)

    R5 = 512
    u2e, st_u2 = pl.pallas_call(
        _e2_body,
        grid=(BN // R5,),
        in_specs=[pl.BlockSpec((R5, K, G), lambda i: (i, 0, 0)),
                  pl.BlockSpec((16, G), lambda i: (0, 0)),
                  pl.BlockSpec((8, G), lambda i: (0, 0)),
                  pl.BlockSpec((G, G), lambda i: (0, 0))],
        out_specs=[pl.BlockSpec((R5, K, G), lambda i: (i, 0, 0)),
                   pl.BlockSpec((16, G), lambda i: (0, 0))],
        out_shape=[jax.ShapeDtypeStruct((BN, K, G), f32),
                   jax.ShapeDtypeStruct((16, G), f32)],
    )(u1e, st_u1, gb_e1, We2)

    out = pl.pallas_call(
        _final_body,
        grid=(BN // R4,),
        in_specs=[pl.BlockSpec((R4, K, G), lambda i: (i, 0, 0)),
                  pl.BlockSpec((16, G), lambda i: (0, 0)),
                  pl.BlockSpec((8, G), lambda i: (0, 0)),
                  pl.BlockSpec((R4, K, C), lambda i: (i, 0, 0)),
                  pl.BlockSpec((R4, C), lambda i: (i, 0)),
                  pl.BlockSpec((16, C), lambda i: (0, 0)),
                  pl.BlockSpec((8, C), lambda i: (0, 0)),
                  pl.BlockSpec((R4, K, C), lambda i: (i, 0, 0)),
                  pl.BlockSpec((G, C), lambda i: (0, 0))],
        out_specs=pl.BlockSpec((R4, C), lambda i: (i, 0)),
        out_shape=jax.ShapeDtypeStruct((BN, C), f32),
    )(u2e, st_u2, gb_e2, ppe, tab, st_pb, gb_pb, ve, erep)

    return out.reshape(B, N, C)


# speedup vs baseline: 1.0200x; 1.0200x over previous
import functools
import jax
import jax.numpy as jnp
from jax import lax
from jax.experimental import pallas as pl
from jax.experimental.pallas import tpu as pltpu
from jax.experimental.pallas import tpu_sc as plsc

B, N, C = 2, 2048, 256
H = 8
G = C // H
K = 16
BN = B * N
E = BN * K
EPS = 1e-5
NW = 32
CH = 128
PER_W = E // NW
TW = 384

_NEG = float('-inf')


def _leaky(x):
    return jnp.where(x >= 0, x, 0.2 * x)


def _proj1_body(xf_ref, wq_ref, wk_ref, wv_ref, wp_ref, y_ref, st_ref):
    i = pl.program_id(0)
    xb = xf_ref[...]

    def mm(w_ref):
        return lax.dot_general(xb, w_ref[...], (((1,), (1,)), ((), ())),
                               preferred_element_type=jnp.float32)

    y = jnp.concatenate([mm(wq_ref), mm(wk_ref), mm(wv_ref), mm(wp_ref)],
                        axis=1)
    y_ref[...] = y
    s = jnp.sum(y, axis=0, keepdims=True)
    s2 = jnp.sum(y * y, axis=0, keepdims=True)
    part = jnp.concatenate(
        [jnp.broadcast_to(s, (8, 4 * C)), jnp.broadcast_to(s2, (8, 4 * C))], axis=0)

    @pl.when(i == 0)
    def _():
        st_ref[...] = jnp.zeros_like(st_ref)

    st_ref[...] += part


def _proj2_body(y_ref, st_ref, gb_ref, we1_ref,
                kn_ref, vn_ref, tab_ref, qe1_ref):
    cnt = jnp.float32(BN)
    mean = st_ref[0:1, :] / cnt
    ex2 = st_ref[8:9, :] / cnt
    var = ex2 - mean * mean
    inv = lax.rsqrt(var + EPS)
    y = y_ref[...]
    act = _leaky((y - mean) * inv * gb_ref[0:1, :] + gb_ref[1:2, :])
    qn = act[:, 0:C]
    kn = act[:, C:2 * C]
    vn = act[:, 2 * C:3 * C]
    kn_ref[...] = kn
    vn_ref[...] = vn
    tab_ref[:, 0:C] = y[:, 3 * C:4 * C]
    we1 = we1_ref[...]

    def mm(a):
        return lax.dot_general(a, we1, (((1,), (1,)), ((), ())),
                               preferred_element_type=jnp.float32)

    qe1_ref[...] = mm(qn)
    tab_ref[:, C:C + G] = mm(kn)
    tab_ref[:, C + G:TW] = jnp.zeros((kn.shape[0], TW - C - G), jnp.float32)


def _topk_body(xb_ref, xa_ref, idx_ref):
    b = pl.program_id(0)
    xb = xb_ref[0]
    xa = xa_ref[0]
    d = lax.dot_general(xb, xa, (((1,), (1,)), ((), ())),
                        preferred_element_type=jnp.float32)
    sqb = jnp.sum(xb * xb, axis=1, keepdims=True)
    sqa = jnp.sum(xa * xa, axis=1)
    nd = 2.0 * d - sqb - sqa[None, :]
    r = nd.shape[0]
    iota = lax.broadcasted_iota(jnp.int32, (r, N), 1)
    cols = []
    for _ in range(K):
        m = jnp.max(nd, axis=1, keepdims=True)
        sel = jnp.where(nd >= m, iota, N)
        a = jnp.min(sel, axis=1, keepdims=True)
        cols.append(a)
        nd = jnp.where(iota == a, _NEG, nd)
    idx = jnp.concatenate(cols, axis=1)
    idx_ref[...] = idx + b * N


def _sc_gather_body(idx_hbm, tab_hbm, out_hbm, idx_v, buf_a, buf_b,
                    sem_a, sem_b):
    wid = lax.axis_index("s") * 2 + lax.axis_index("c")
    wbase = pl.multiple_of(wid * PER_W, CH)
    nch = PER_W // CH

    pltpu.sync_copy(idx_hbm.at[pl.ds(pl.multiple_of(wid * nch, 8), nch)], idx_v)
    pltpu.async_copy(tab_hbm.at[idx_v.at[0]], buf_a, sem_a)

    def body(j, carry):
        i0 = 2 * j
        pltpu.async_copy(tab_hbm.at[idx_v.at[i0 + 1]], buf_b, sem_b)
        pltpu.make_async_copy(tab_hbm.at[idx_v.at[i0]], buf_a, sem_a).wait()
        pltpu.sync_copy(buf_a, out_hbm.at[pl.ds(wbase + i0 * CH, CH)])

        @pl.when(i0 + 2 < nch)
        def _():
            pltpu.async_copy(tab_hbm.at[idx_v.at[i0 + 2]], buf_a, sem_a)

        pltpu.make_async_copy(tab_hbm.at[idx_v.at[i0 + 1]], buf_b, sem_b).wait()
        pltpu.sync_copy(buf_b, out_hbm.at[pl.ds(wbase + (i0 + 1) * CH, CH)])
        return carry

    lax.fori_loop(0, nch // 2, body, 0)


def _sc_gatherk_body(idx_hbm, tab_hbm, ppe_hbm, ke1e_hbm, idx_v, buf_a, buf_b,
                     sem_a, sem_b):
    wid = lax.axis_index("s") * 2 + lax.axis_index("c")
    wbase = pl.multiple_of(wid * PER_W, CH)
    nch = PER_W // CH

    pltpu.sync_copy(idx_hbm.at[pl.ds(pl.multiple_of(wid * nch, 8), nch)], idx_v)
    pltpu.async_copy(tab_hbm.at[idx_v.at[0]], buf_a, sem_a)

    def wb(buf, i):
        pltpu.sync_copy(buf.at[:, pl.ds(0, C)],
                        ppe_hbm.at[pl.ds(wbase + i * CH, CH)])
        pltpu.sync_copy(buf.at[:, pl.ds(C, 128)],
                        ke1e_hbm.at[pl.ds(wbase + i * CH, CH)])

    def body(j, carry):
        i0 = 2 * j
        pltpu.async_copy(tab_hbm.at[idx_v.at[i0 + 1]], buf_b, sem_b)
        pltpu.make_async_copy(tab_hbm.at[idx_v.at[i0]], buf_a, sem_a).wait()
        wb(buf_a, i0)

        @pl.when(i0 + 2 < nch)
        def _():
            pltpu.async_copy(tab_hbm.at[idx_v.at[i0 + 2]], buf_a, sem_a)

        pltpu.make_async_copy(tab_hbm.at[idx_v.at[i0 + 1]], buf_b, sem_b).wait()
        wb(buf_b, i0 + 1)
        return carry

    lax.fori_loop(0, nch // 2, body, 0)


def _pbstat_body(ppe_ref, pp_ref, st_ref):
    i = pl.program_id(0)
    r = pp_ref.shape[0]
    y = ppe_ref[...].reshape(r, K, C) - pp_ref[...][:, None, :]
    s = jnp.sum(y, axis=(0, 1))[None, :]
    s2 = jnp.sum(y * y, axis=(0, 1))[None, :]
    part = jnp.concatenate(
        [jnp.broadcast_to(s, (8, C)), jnp.broadcast_to(s2, (8, C))], axis=0)

    @pl.when(i == 0)
    def _():
        st_ref[...] = jnp.zeros_like(st_ref)

    st_ref[...] += part


def _e1_body(ppe_ref, pp_ref, stpb_ref, gbpb_ref, qe1_ref, ke1e_ref, we1_ref,
             u1_ref, st_ref):
    i = pl.program_id(0)
    cnt = jnp.float32(E)
    mean = stpb_ref[0:1, :] / cnt
    ex2 = stpb_ref[8:9, :] / cnt
    inv = lax.rsqrt(ex2 - mean * mean + EPS)
    r = pp_ref.shape[0]
    y = ppe_ref[...].reshape(r, K, C) - pp_ref[...][:, None, :]
    posb = _leaky((y - mean[None, :, :]) * inv[None, :, :]
                  * gbpb_ref[0:1, :][None, :, :] + gbpb_ref[1:2, :][None, :, :])
    u1 = lax.dot_general(posb.reshape(r * K, C), we1_ref[...],
                         (((1,), (1,)), ((), ())),
                         preferred_element_type=jnp.float32).reshape(r, K, G)
    u1 = u1 + ke1e_ref[...].reshape(r, K, 128)[:, :, 0:G] \
        - qe1_ref[...][:, None, :]
    u1_ref[...] = u1
    s = jnp.sum(u1, axis=(0, 1))[None, :]
    s2 = jnp.sum(u1 * u1, axis=(0, 1))[None, :]
    part = jnp.concatenate(
        [jnp.broadcast_to(s, (8, G)), jnp.broadcast_to(s2, (8, G))], axis=0)

    @pl.when(i == 0)
    def _():
        st_ref[...] = jnp.zeros_like(st_ref)

    st_ref[...] += part


def _e2_body(u1_ref, st1_ref, gbe1_ref, we2_ref, u2_ref, st_ref):
    i = pl.program_id(0)
    cnt = jnp.float32(E)
    mean = st1_ref[0:1, :] / cnt
    ex2 = st1_ref[8:9, :] / cnt
    inv = lax.rsqrt(ex2 - mean * mean + EPS)
    t1 = _leaky((u1_ref[...] - mean[None, :, :]) * inv[None, :, :]
                * gbe1_ref[0:1, :][None, :, :] + gbe1_ref[1:2, :][None, :, :])
    r = t1.shape[0]
    u2 = lax.dot_general(t1.reshape(r * K, G), we2_ref[...],
                         (((1,), (1,)), ((), ())),
                         preferred_element_type=jnp.float32).reshape(r, K, G)
    u2_ref[...] = u2
    s = jnp.sum(u2, axis=(0, 1))[None, :]
    s2 = jnp.sum(u2 * u2, axis=(0, 1))[None, :]
    part = jnp.concatenate(
        [jnp.broadcast_to(s, (8, G)), jnp.broadcast_to(s2, (8, G))], axis=0)

    @pl.when(i == 0)
    def _():
        st_ref[...] = jnp.zeros_like(st_ref)

    st_ref[...] += part


def _final_body(u2_ref, st2_ref, gbe2_ref, ppe_ref, pp_ref, stpb_ref,
                gbpb_ref, ve_ref, erep_ref, out_ref):
    cnt = jnp.float32(E)
    mean2 = st2_ref[0:1, :] / cnt
    ex22 = st2_ref[8:9, :] / cnt
    inv2 = lax.rsqrt(ex22 - mean2 * mean2 + EPS)
    t2 = _leaky((u2_ref[...] - mean2[None, :, :]) * inv2[None, :, :]
                * gbe2_ref[0:1, :][None, :, :] + gbe2_ref[1:2, :][None, :, :])
    mx = jnp.max(t2, axis=1, keepdims=True)
    exv = jnp.exp(t2 - mx)
    attn = exv / jnp.sum(exv, axis=1, keepdims=True)
    r = attn.shape[0]
    attn_c = jnp.dot(attn.reshape(r * K, G), erep_ref[...],
                     preferred_element_type=jnp.float32).reshape(r, K, C)
    meanp = stpb_ref[0:1, :] / cnt
    ex2p = stpb_ref[8:9, :] / cnt
    invp = lax.rsqrt(ex2p - meanp * meanp + EPS)
    y = ppe_ref[...].reshape(r, K, C) - pp_ref[...][:, None, :]
    posb = _leaky((y - meanp[None, :, :]) * invp[None, :, :]
                  * gbpb_ref[0:1, :][None, :, :] + gbpb_ref[1:2, :][None, :, :])
    value = ve_ref[...].reshape(r, K, C) + posb
    out_ref[...] = jnp.sum(value * attn_c, axis=1)


def kernel(x, Wq, bq, gq, betaq, Wk, bk, gk, betak, Wv, bv, gv, betav,
           Wpb, bpb, gpb, betapb, We1, be1, ge1, betae1, We2, be2, ge2, betae2):
    f32 = jnp.float32
    xf = x.reshape(BN, C)

    gam = jnp.concatenate([gq, gk, gv, jnp.ones((C,), f32)])
    bet = jnp.concatenate([betaq, betak, betav, jnp.zeros((C,), f32)])
    gb_qkv = jnp.zeros((8, 4 * C), f32).at[0].set(gam).at[1].set(bet)
    gb_pb = jnp.zeros((8, C), f32).at[0].set(gpb).at[1].set(betapb)
    gb_e1 = jnp.zeros((8, G), f32).at[0].set(ge1).at[1].set(betae1)
    gb_e2 = jnp.zeros((8, G), f32).at[0].set(ge2).at[1].set(betae2)
    erep = jnp.repeat(jnp.eye(G, dtype=f32), H, axis=1).reshape(G, C)

    R1 = 512
    wspec = pl.BlockSpec((C, C), lambda i: (0, 0))
    y_all, st_qkv = pl.pallas_call(
        _proj1_body,
        grid=(BN // R1,),
        in_specs=[pl.BlockSpec((R1, C), lambda i: (i, 0)),
                  wspec, wspec, wspec, wspec],
        out_specs=[pl.BlockSpec((R1, 4 * C), lambda i: (i, 0)),
                   pl.BlockSpec((16, 4 * C), lambda i: (0, 0))],
        out_shape=[jax.ShapeDtypeStruct((BN, 4 * C), f32),
                   jax.ShapeDtypeStruct((16, 4 * C), f32)],
    )(xf, Wq, Wk, Wv, Wpb)

    R2 = 512
    kn, vn, tab, qe1 = pl.pallas_call(
        _proj2_body,
        grid=(BN // R2,),
        in_specs=[pl.BlockSpec((R2, 4 * C), lambda i: (i, 0)),
                  pl.BlockSpec((16, 4 * C), lambda i: (0, 0)),
                  pl.BlockSpec((8, 4 * C), lambda i: (0, 0)),
                  pl.BlockSpec((G, C), lambda i: (0, 0))],
        out_specs=[pl.BlockSpec((R2, C), lambda i: (i, 0)),
                   pl.BlockSpec((R2, C), lambda i: (i, 0)),
                   pl.BlockSpec((R2, TW), lambda i: (i, 0)),
                   pl.BlockSpec((R2, G), lambda i: (i, 0))],
        out_shape=[jax.ShapeDtypeStruct((BN, C), f32),
                   jax.ShapeDtypeStruct((BN, C), f32),
                   jax.ShapeDtypeStruct((BN, TW), f32),
                   jax.ShapeDtypeStruct((BN, G), f32)],
    )(y_all, st_qkv, gb_qkv, We1)

    R3 = 256

    def _run_topk(feat):
        return pl.pallas_call(
            _topk_body,
            grid=(B, N // R3),
            in_specs=[pl.BlockSpec((1, R3, C), lambda b, j: (b, j, 0)),
                      pl.BlockSpec((1, N, C), lambda b, j: (b, 0, 0))],
            out_specs=pl.BlockSpec((R3, K), lambda b, j: (b * (N // R3) + j, 0)),
            out_shape=jax.ShapeDtypeStruct((BN, K), jnp.int32),
        )(feat, feat)

    idxg = _run_topk(kn.reshape(B, N, C))
    idxgv = _run_topk(vn.reshape(B, N, C))

    mesh = plsc.VectorSubcoreMesh(core_axis_name="c", subcore_axis_name="s")

    def _scratch(width):
        return [pltpu.VMEM((E // NW // CH, CH), jnp.int32),
                pltpu.VMEM((CH, width), f32),
                pltpu.VMEM((CH, width), f32),
                pltpu.SemaphoreType.DMA,
                pltpu.SemaphoreType.DMA]

    gather_k = functools.partial(
        pl.kernel, mesh=mesh,
        out_type=[jax.ShapeDtypeStruct((E, C), f32),
                  jax.ShapeDtypeStruct((E, 128), f32)],
        scratch_types=_scratch(TW),
    )(_sc_gatherk_body)
    ppe_f, ke1e_f = gather_k(idxg.reshape(E // CH, CH), tab)

    gather_v = functools.partial(
        pl.kernel, mesh=mesh,
        out_type=jax.ShapeDtypeStruct((E, C), f32),
        scratch_types=_scratch(C),
    )(_sc_gather_body)
    ve_f = gather_v(idxgv.reshape(E // CH, CH), vn)

    R4 = 256
    st_pb = pl.pallas_call(
        _pbstat_body,
        grid=(BN // R4,),
        in_specs=[pl.BlockSpec((R4 * K, C), lambda i: (i, 0)),
                  pl.BlockSpec((R4, C), lambda i: (i, 0))],
        out_specs=pl.BlockSpec((16, C), lambda i: (0, 0)),
        out_shape=jax.ShapeDtypeStruct((16, C), f32),
    )(ppe_f, tab)

    u1e, st_u1 = pl.pallas_call(
        _e1_body,
        grid=(BN // R4,),
        in_specs=[pl.BlockSpec((R4 * K, C), lambda i: (i, 0)),
                  pl.BlockSpec((R4, C), lambda i: (i, 0)),
                  pl.BlockSpec((16, C), lambda i: (0, 0)),
                  pl.BlockSpec((8, C), lambda i: (0, 0)),
                  pl.BlockSpec((R4, G), lambda i: (i, 0)),
                  pl.BlockSpec((R4 * K, 128), lambda i: (i, 0)),
                  pl.BlockSpec((G, C), lambda i: (0, 0))],
        out_specs=[pl.BlockSpec((R4, K, G), lambda i: (i, 0, 0)),
                   pl.BlockSpec((16, G), lambda i: (0, 0))],
        out_shape=[jax.ShapeDtypeStruct((BN, K, G), f32),
                   jax.ShapeDtypeStruct((16, G), f32)],
    )(ppe_f, tab, st_pb, gb_pb, qe1, ke1e_f, We1)

    R5 = 512
    u2e, st_u2 = pl.pallas_call(
        _e2_body,
        grid=(BN // R5,),
        in_specs=[pl.BlockSpec((R5, K, G), lambda i: (i, 0, 0)),
                  pl.BlockSpec((16, G), lambda i: (0, 0)),
                  pl.BlockSpec((8, G), lambda i: (0, 0)),
                  pl.BlockSpec((G, G), lambda i: (0, 0))],
        out_specs=[pl.BlockSpec((R5, K, G), lambda i: (i, 0, 0)),
                   pl.BlockSpec((16, G), lambda i: (0, 0))],
        out_shape=[jax.ShapeDtypeStruct((BN, K, G), f32),
                   jax.ShapeDtypeStruct((16, G), f32)],
    )(u1e, st_u1, gb_e1, We2)

    out = pl.pallas_call(
        _final_body,
        grid=(BN // R4,),
        in_specs=[pl.BlockSpec((R4, K, G), lambda i: (i, 0, 0)),
                  pl.BlockSpec((16, G), lambda i: (0, 0)),
                  pl.BlockSpec((8, G), lambda i: (0, 0)),
                  pl.BlockSpec((R4 * K, C), lambda i: (i, 0)),
                  pl.BlockSpec((R4, C), lambda i: (i, 0)),
                  pl.BlockSpec((16, C), lambda i: (0, 0)),
                  pl.BlockSpec((8, C), lambda i: (0, 0)),
                  pl.BlockSpec((R4 * K, C), lambda i: (i, 0)),
                  pl.BlockSpec((G, C), lambda i: (0, 0))],
        out_specs=pl.BlockSpec((R4, C), lambda i: (i, 0)),
        out_shape=jax.ShapeDtypeStruct((BN, C), f32),
    )(u2e, st_u2, gb_e2, ppe_f, tab, st_pb, gb_pb, ve_f, erep)

    return out.reshape(B, N, C)


# speedup vs baseline: 1.0624x; 1.0416x over previous
import functools
import jax
import jax.numpy as jnp
from jax import lax
from jax.experimental import pallas as pl
from jax.experimental.pallas import tpu as pltpu
from jax.experimental.pallas import tpu_sc as plsc

B, N, C = 2, 2048, 256
H = 8
G = C // H
K = 16
BN = B * N
E = BN * K
EPS = 1e-5
NW = 32
CH = 128
PER_W = E // NW
TW = 384

_NEG = float('-inf')


def _leaky(x):
    return jnp.where(x >= 0, x, 0.2 * x)


def _proj1_body(xf_ref, wq_ref, wk_ref, wv_ref, wp_ref, y_ref, st_ref):
    i = pl.program_id(0)
    xb = xf_ref[...]

    def mm(w_ref):
        return lax.dot_general(xb, w_ref[...], (((1,), (1,)), ((), ())),
                               preferred_element_type=jnp.float32)

    y = jnp.concatenate([mm(wq_ref), mm(wk_ref), mm(wv_ref), mm(wp_ref)],
                        axis=1)
    y_ref[...] = y
    s = jnp.sum(y, axis=0, keepdims=True)
    s2 = jnp.sum(y * y, axis=0, keepdims=True)
    part = jnp.concatenate(
        [jnp.broadcast_to(s, (8, 4 * C)), jnp.broadcast_to(s2, (8, 4 * C))], axis=0)

    @pl.when(i == 0)
    def _():
        st_ref[...] = jnp.zeros_like(st_ref)

    st_ref[...] += part


def _proj2_body(y_ref, st_ref, gb_ref, we1_ref,
                kn_ref, vn_ref, tab_ref, qe1_ref):
    cnt = jnp.float32(BN)
    mean = st_ref[0:1, :] / cnt
    ex2 = st_ref[8:9, :] / cnt
    var = ex2 - mean * mean
    inv = lax.rsqrt(var + EPS)
    y = y_ref[...]
    act = _leaky((y - mean) * inv * gb_ref[0:1, :] + gb_ref[1:2, :])
    qn = act[:, 0:C]
    kn = act[:, C:2 * C]
    vn = act[:, 2 * C:3 * C]
    kn_ref[...] = kn
    vn_ref[...] = vn
    tab_ref[:, 0:C] = y[:, 3 * C:4 * C]
    we1 = we1_ref[...]

    def mm(a):
        return lax.dot_general(a, we1, (((1,), (1,)), ((), ())),
                               preferred_element_type=jnp.float32)

    qe1_ref[...] = mm(qn)
    tab_ref[:, C:C + G] = mm(kn)
    tab_ref[:, C + G:TW] = jnp.zeros((kn.shape[0], TW - C - G), jnp.float32)


def _topk_body(xb_ref, xa_ref, idx_ref):
    b = pl.program_id(0)
    xb = xb_ref[0]
    xa = xa_ref[0]
    d = lax.dot_general(xb, xa, (((1,), (1,)), ((), ())),
                        preferred_element_type=jnp.float32)
    sqb = jnp.sum(xb * xb, axis=1, keepdims=True)
    sqa = jnp.sum(xa * xa, axis=1)
    nd = 2.0 * d - sqb - sqa[None, :]
    r = nd.shape[0]
    iota = lax.broadcasted_iota(jnp.int32, (r, N), 1)
    cols = []
    for _ in range(K):
        m = jnp.max(nd, axis=1, keepdims=True)
        sel = jnp.where(nd >= m, iota, N)
        a = jnp.min(sel, axis=1, keepdims=True)
        cols.append(a)
        nd = jnp.where(iota == a, _NEG, nd)
    idx = jnp.concatenate(cols, axis=1)
    idx_ref[...] = idx + b * N


def _sc_gather_body(idx_hbm, tab_hbm, out_hbm, idx_v, buf_a, buf_b,
                    sem_a, sem_b):
    wid = lax.axis_index("s") * 2 + lax.axis_index("c")
    wbase = pl.multiple_of(wid * PER_W, CH)
    nch = PER_W // CH

    pltpu.sync_copy(idx_hbm.at[pl.ds(pl.multiple_of(wid * nch, 8), nch)], idx_v)
    pltpu.async_copy(tab_hbm.at[idx_v.at[0]], buf_a, sem_a)

    def body(j, carry):
        i0 = 2 * j
        pltpu.async_copy(tab_hbm.at[idx_v.at[i0 + 1]], buf_b, sem_b)
        pltpu.make_async_copy(tab_hbm.at[idx_v.at[i0]], buf_a, sem_a).wait()
        pltpu.sync_copy(buf_a, out_hbm.at[pl.ds(wbase + i0 * CH, CH)])

        @pl.when(i0 + 2 < nch)
        def _():
            pltpu.async_copy(tab_hbm.at[idx_v.at[i0 + 2]], buf_a, sem_a)

        pltpu.make_async_copy(tab_hbm.at[idx_v.at[i0 + 1]], buf_b, sem_b).wait()
        pltpu.sync_copy(buf_b, out_hbm.at[pl.ds(wbase + (i0 + 1) * CH, CH)])
        return carry

    lax.fori_loop(0, nch // 2, body, 0)


NV = C // 16


def _sc_gatherk_body(idx_hbm, tab_hbm, ppe_hbm, ke1e_hbm, stp_hbm,
                     idx_v, buf_a, buf_b, nodebuf, stbuf, sem_a, sem_b):
    wid = lax.axis_index("s") * 2 + lax.axis_index("c")
    wbase = pl.multiple_of(wid * PER_W, CH)
    nch = PER_W // CH

    pltpu.sync_copy(idx_hbm.at[pl.ds(pl.multiple_of(wid * nch, 8), nch)], idx_v)
    pltpu.async_copy(tab_hbm.at[idx_v.at[0]], buf_a, sem_a)

    def wb(buf, i):
        pltpu.sync_copy(buf.at[:, pl.ds(0, C)],
                        ppe_hbm.at[pl.ds(wbase + i * CH, CH)])
        pltpu.sync_copy(buf.at[:, pl.ds(C, 128)],
                        ke1e_hbm.at[pl.ds(wbase + i * CH, CH)])

    def accum(buf, i, acc):
        node0 = pl.multiple_of((wbase + i * CH) // K, 8)
        pltpu.sync_copy(tab_hbm.at[pl.ds(node0, CH // K)], nodebuf)

        def nbody(nn, acc):
            pv = [nodebuf[nn, pl.ds(16 * v, 16)] for v in range(NV)]

            def ebody(k, acc):
                s, q = acc
                e = nn * K + k
                ns, nq = [], []
                for v in range(NV):
                    d = buf[e, pl.ds(16 * v, 16)] - pv[v]
                    ns.append(s[v] + d)
                    nq.append(q[v] + d * d)
                return (tuple(ns), tuple(nq))

            return lax.fori_loop(0, K, ebody, acc)

        return lax.fori_loop(0, CH // K, nbody, acc)

    zero = jnp.zeros((16,), jnp.float32)
    acc0 = (tuple([zero] * NV), tuple([zero] * NV))

    def body(j, acc):
        i0 = 2 * j
        pltpu.async_copy(tab_hbm.at[idx_v.at[i0 + 1]], buf_b, sem_b)
        pltpu.make_async_copy(tab_hbm.at[idx_v.at[i0]], buf_a, sem_a).wait()
        wb(buf_a, i0)
        acc = accum(buf_a, i0, acc)

        @pl.when(i0 + 2 < nch)
        def _():
            pltpu.async_copy(tab_hbm.at[idx_v.at[i0 + 2]], buf_a, sem_a)

        pltpu.make_async_copy(tab_hbm.at[idx_v.at[i0 + 1]], buf_b, sem_b).wait()
        wb(buf_b, i0 + 1)
        acc = accum(buf_b, i0 + 1, acc)
        return acc

    s, q = lax.fori_loop(0, nch // 2, body, acc0)
    for v in range(NV):
        stbuf[0, pl.ds(16 * v, 16)] = s[v]
        stbuf[1, pl.ds(16 * v, 16)] = q[v]
        for r in range(2, 8):
            stbuf[r, pl.ds(16 * v, 16)] = zero
    pltpu.sync_copy(stbuf, stp_hbm.at[wid])


def _e1_body(ppe_ref, pp_ref, stpb_ref, gbpb_ref, qe1_ref, ke1e_ref, we1_ref,
             u1_ref, st_ref):
    i = pl.program_id(0)
    cnt = jnp.float32(E)
    mean = jnp.sum(stpb_ref[:, 0, :], axis=0)[None, :] / cnt
    ex2 = jnp.sum(stpb_ref[:, 1, :], axis=0)[None, :] / cnt
    inv = lax.rsqrt(ex2 - mean * mean + EPS)
    r = pp_ref.shape[0]
    y = ppe_ref[...].reshape(r, K, C) - pp_ref[...][:, None, :]
    posb = _leaky((y - mean[None, :, :]) * inv[None, :, :]
                  * gbpb_ref[0:1, :][None, :, :] + gbpb_ref[1:2, :][None, :, :])
    u1 = lax.dot_general(posb.reshape(r * K, C), we1_ref[...],
                         (((1,), (1,)), ((), ())),
                         preferred_element_type=jnp.float32).reshape(r, K, G)
    u1 = u1 + ke1e_ref[...].reshape(r, K, 128)[:, :, 0:G] \
        - qe1_ref[...][:, None, :]
    u1_ref[...] = u1
    s = jnp.sum(u1, axis=(0, 1))[None, :]
    s2 = jnp.sum(u1 * u1, axis=(0, 1))[None, :]
    part = jnp.concatenate(
        [jnp.broadcast_to(s, (8, G)), jnp.broadcast_to(s2, (8, G))], axis=0)

    @pl.when(i == 0)
    def _():
        st_ref[...] = jnp.zeros_like(st_ref)

    st_ref[...] += part


def _e2_body(u1_ref, st1_ref, gbe1_ref, we2_ref, u2_ref, st_ref):
    i = pl.program_id(0)
    cnt = jnp.float32(E)
    mean = st1_ref[0:1, :] / cnt
    ex2 = st1_ref[8:9, :] / cnt
    inv = lax.rsqrt(ex2 - mean * mean + EPS)
    t1 = _leaky((u1_ref[...] - mean[None, :, :]) * inv[None, :, :]
                * gbe1_ref[0:1, :][None, :, :] + gbe1_ref[1:2, :][None, :, :])
    r = t1.shape[0]
    u2 = lax.dot_general(t1.reshape(r * K, G), we2_ref[...],
                         (((1,), (1,)), ((), ())),
                         preferred_element_type=jnp.float32).reshape(r, K, G)
    u2_ref[...] = u2
    s = jnp.sum(u2, axis=(0, 1))[None, :]
    s2 = jnp.sum(u2 * u2, axis=(0, 1))[None, :]
    part = jnp.concatenate(
        [jnp.broadcast_to(s, (8, G)), jnp.broadcast_to(s2, (8, G))], axis=0)

    @pl.when(i == 0)
    def _():
        st_ref[...] = jnp.zeros_like(st_ref)

    st_ref[...] += part


def _final_body(u2_ref, st2_ref, gbe2_ref, ppe_ref, pp_ref, stpb_ref,
                gbpb_ref, ve_ref, erep_ref, out_ref):
    cnt = jnp.float32(E)
    mean2 = st2_ref[0:1, :] / cnt
    ex22 = st2_ref[8:9, :] / cnt
    inv2 = lax.rsqrt(ex22 - mean2 * mean2 + EPS)
    t2 = _leaky((u2_ref[...] - mean2[None, :, :]) * inv2[None, :, :]
                * gbe2_ref[0:1, :][None, :, :] + gbe2_ref[1:2, :][None, :, :])
    mx = jnp.max(t2, axis=1, keepdims=True)
    exv = jnp.exp(t2 - mx)
    attn = exv / jnp.sum(exv, axis=1, keepdims=True)
    r = attn.shape[0]
    attn_c = jnp.dot(attn.reshape(r * K, G), erep_ref[...],
                     preferred_element_type=jnp.float32).reshape(r, K, C)
    meanp = jnp.sum(stpb_ref[:, 0, :], axis=0)[None, :] / cnt
    ex2p = jnp.sum(stpb_ref[:, 1, :], axis=0)[None, :] / cnt
    invp = lax.rsqrt(ex2p - meanp * meanp + EPS)
    y = ppe_ref[...].reshape(r, K, C) - pp_ref[...][:, None, :]
    posb = _leaky((y - meanp[None, :, :]) * invp[None, :, :]
                  * gbpb_ref[0:1, :][None, :, :] + gbpb_ref[1:2, :][None, :, :])
    value = ve_ref[...].reshape(r, K, C) + posb
    out_ref[...] = jnp.sum(value * attn_c, axis=1)


def kernel(x, Wq, bq, gq, betaq, Wk, bk, gk, betak, Wv, bv, gv, betav,
           Wpb, bpb, gpb, betapb, We1, be1, ge1, betae1, We2, be2, ge2, betae2):
    f32 = jnp.float32
    xf = x.reshape(BN, C)

    gam = jnp.concatenate([gq, gk, gv, jnp.ones((C,), f32)])
    bet = jnp.concatenate([betaq, betak, betav, jnp.zeros((C,), f32)])
    gb_qkv = jnp.zeros((8, 4 * C), f32).at[0].set(gam).at[1].set(bet)
    gb_pb = jnp.zeros((8, C), f32).at[0].set(gpb).at[1].set(betapb)
    gb_e1 = jnp.zeros((8, G), f32).at[0].set(ge1).at[1].set(betae1)
    gb_e2 = jnp.zeros((8, G), f32).at[0].set(ge2).at[1].set(betae2)
    erep = jnp.repeat(jnp.eye(G, dtype=f32), H, axis=1).reshape(G, C)

    R1 = 512
    wspec = pl.BlockSpec((C, C), lambda i: (0, 0))
    y_all, st_qkv = pl.pallas_call(
        _proj1_body,
        grid=(BN // R1,),
        in_specs=[pl.BlockSpec((R1, C), lambda i: (i, 0)),
                  wspec, wspec, wspec, wspec],
        out_specs=[pl.BlockSpec((R1, 4 * C), lambda i: (i, 0)),
                   pl.BlockSpec((16, 4 * C), lambda i: (0, 0))],
        out_shape=[jax.ShapeDtypeStruct((BN, 4 * C), f32),
                   jax.ShapeDtypeStruct((16, 4 * C), f32)],
    )(xf, Wq, Wk, Wv, Wpb)

    R2 = 512
    kn, vn, tab, qe1 = pl.pallas_call(
        _proj2_body,
        grid=(BN // R2,),
        in_specs=[pl.BlockSpec((R2, 4 * C), lambda i: (i, 0)),
                  pl.BlockSpec((16, 4 * C), lambda i: (0, 0)),
                  pl.BlockSpec((8, 4 * C), lambda i: (0, 0)),
                  pl.BlockSpec((G, C), lambda i: (0, 0))],
        out_specs=[pl.BlockSpec((R2, C), lambda i: (i, 0)),
                   pl.BlockSpec((R2, C), lambda i: (i, 0)),
                   pl.BlockSpec((R2, TW), lambda i: (i, 0)),
                   pl.BlockSpec((R2, G), lambda i: (i, 0))],
        out_shape=[jax.ShapeDtypeStruct((BN, C), f32),
                   jax.ShapeDtypeStruct((BN, C), f32),
                   jax.ShapeDtypeStruct((BN, TW), f32),
                   jax.ShapeDtypeStruct((BN, G), f32)],
    )(y_all, st_qkv, gb_qkv, We1)

    R3 = 256

    def _run_topk(feat):
        return pl.pallas_call(
            _topk_body,
            grid=(B, N // R3),
            in_specs=[pl.BlockSpec((1, R3, C), lambda b, j: (b, j, 0)),
                      pl.BlockSpec((1, N, C), lambda b, j: (b, 0, 0))],
            out_specs=pl.BlockSpec((R3, K), lambda b, j: (b * (N // R3) + j, 0)),
            out_shape=jax.ShapeDtypeStruct((BN, K), jnp.int32),
        )(feat, feat)

    idxg = _run_topk(kn.reshape(B, N, C))
    idxgv = _run_topk(vn.reshape(B, N, C))

    mesh = plsc.VectorSubcoreMesh(core_axis_name="c", subcore_axis_name="s")

    def _scratch(width):
        return [pltpu.VMEM((E // NW // CH, CH), jnp.int32),
                pltpu.VMEM((CH, width), f32),
                pltpu.VMEM((CH, width), f32),
                pltpu.SemaphoreType.DMA,
                pltpu.SemaphoreType.DMA]

    gather_k = functools.partial(
        pl.kernel, mesh=mesh,
        out_type=[jax.ShapeDtypeStruct((E, C), f32),
                  jax.ShapeDtypeStruct((E, 128), f32),
                  jax.ShapeDtypeStruct((NW, 8, C), f32)],
        scratch_types=[pltpu.VMEM((E // NW // CH, CH), jnp.int32),
                       pltpu.VMEM((CH, TW), f32),
                       pltpu.VMEM((CH, TW), f32),
                       pltpu.VMEM((CH // K, TW), f32),
                       pltpu.VMEM((8, C), f32),
                       pltpu.SemaphoreType.DMA,
                       pltpu.SemaphoreType.DMA],
    )(_sc_gatherk_body)
    ppe_f, ke1e_f, stp = gather_k(idxg.reshape(E // CH, CH), tab)

    gather_v = functools.partial(
        pl.kernel, mesh=mesh,
        out_type=jax.ShapeDtypeStruct((E, C), f32),
        scratch_types=_scratch(C),
    )(_sc_gather_body)
    ve_f = gather_v(idxgv.reshape(E // CH, CH), vn)

    R4 = 256
    u1e, st_u1 = pl.pallas_call(
        _e1_body,
        grid=(BN // R4,),
        in_specs=[pl.BlockSpec((R4 * K, C), lambda i: (i, 0)),
                  pl.BlockSpec((R4, C), lambda i: (i, 0)),
                  pl.BlockSpec((NW, 8, C), lambda i: (0, 0, 0)),
                  pl.BlockSpec((8, C), lambda i: (0, 0)),
                  pl.BlockSpec((R4, G), lambda i: (i, 0)),
                  pl.BlockSpec((R4 * K, 128), lambda i: (i, 0)),
                  pl.BlockSpec((G, C), lambda i: (0, 0))],
        out_specs=[pl.BlockSpec((R4, K, G), lambda i: (i, 0, 0)),
                   pl.BlockSpec((16, G), lambda i: (0, 0))],
        out_shape=[jax.ShapeDtypeStruct((BN, K, G), f32),
                   jax.ShapeDtypeStruct((16, G), f32)],
    )(ppe_f, tab, stp, gb_pb, qe1, ke1e_f, We1)

    R5 = 512
    u2e, st_u2 = pl.pallas_call(
        _e2_body,
        grid=(BN // R5,),
        in_specs=[pl.BlockSpec((R5, K, G), lambda i: (i, 0, 0)),
                  pl.BlockSpec((16, G), lambda i: (0, 0)),
                  pl.BlockSpec((8, G), lambda i: (0, 0)),
                  pl.BlockSpec((G, G), lambda i: (0, 0))],
        out_specs=[pl.BlockSpec((R5, K, G), lambda i: (i, 0, 0)),
                   pl.BlockSpec((16, G), lambda i: (0, 0))],
        out_shape=[jax.ShapeDtypeStruct((BN, K, G), f32),
                   jax.ShapeDtypeStruct((16, G), f32)],
    )(u1e, st_u1, gb_e1, We2)

    out = pl.pallas_call(
        _final_body,
        grid=(BN // R4,),
        in_specs=[pl.BlockSpec((R4, K, G), lambda i: (i, 0, 0)),
                  pl.BlockSpec((16, G), lambda i: (0, 0)),
                  pl.BlockSpec((8, G), lambda i: (0, 0)),
                  pl.BlockSpec((R4 * K, C), lambda i: (i, 0)),
                  pl.BlockSpec((R4, C), lambda i: (i, 0)),
                  pl.BlockSpec((NW, 8, C), lambda i: (0, 0, 0)),
                  pl.BlockSpec((8, C), lambda i: (0, 0)),
                  pl.BlockSpec((R4 * K, C), lambda i: (i, 0)),
                  pl.BlockSpec((G, C), lambda i: (0, 0))],
        out_specs=pl.BlockSpec((R4, C), lambda i: (i, 0)),
        out_shape=jax.ShapeDtypeStruct((BN, C), f32),
    )(u2e, st_u2, gb_e2, ppe_f, tab, stp, gb_pb, ve_f, erep)

    return out.reshape(B, N, C)


# speedup vs baseline: 1.0631x; 1.0007x over previous
import functools
import jax
import jax.numpy as jnp
from jax import lax
from jax.experimental import pallas as pl
from jax.experimental.pallas import tpu as pltpu
from jax.experimental.pallas import tpu_sc as plsc

B, N, C = 2, 2048, 256
H = 8
G = C // H
K = 16
BN = B * N
E = BN * K
EPS = 1e-5
NW = 32
CH = 128
PER_W = E // NW
TW = 384

_NEG = float('-inf')


def _leaky(x):
    return jnp.where(x >= 0, x, 0.2 * x)


def _proj1_body(xf_ref, wq_ref, wk_ref, wv_ref, wp_ref, y_ref, st_ref):
    i = pl.program_id(0)
    xb = xf_ref[...]

    def mm(w_ref):
        return lax.dot_general(xb, w_ref[...], (((1,), (1,)), ((), ())),
                               preferred_element_type=jnp.float32)

    y = jnp.concatenate([mm(wq_ref), mm(wk_ref), mm(wv_ref), mm(wp_ref)],
                        axis=1)
    y_ref[...] = y
    s = jnp.sum(y, axis=0, keepdims=True)
    s2 = jnp.sum(y * y, axis=0, keepdims=True)
    part = jnp.concatenate(
        [jnp.broadcast_to(s, (8, 4 * C)), jnp.broadcast_to(s2, (8, 4 * C))], axis=0)

    @pl.when(i == 0)
    def _():
        st_ref[...] = jnp.zeros_like(st_ref)

    st_ref[...] += part


def _proj2_body(y_ref, st_ref, gb_ref, we1_ref,
                kn_ref, vn_ref, tab_ref, qe1_ref):
    cnt = jnp.float32(BN)
    mean = st_ref[0:1, :] / cnt
    ex2 = st_ref[8:9, :] / cnt
    var = ex2 - mean * mean
    inv = lax.rsqrt(var + EPS)
    y = y_ref[...]
    act = _leaky((y - mean) * inv * gb_ref[0:1, :] + gb_ref[1:2, :])
    qn = act[:, 0:C]
    kn = act[:, C:2 * C]
    vn = act[:, 2 * C:3 * C]
    kn_ref[...] = kn
    vn_ref[...] = vn
    tab_ref[:, 0:C] = y[:, 3 * C:4 * C]
    we1 = we1_ref[...]

    def mm(a):
        return lax.dot_general(a, we1, (((1,), (1,)), ((), ())),
                               preferred_element_type=jnp.float32)

    qe1_ref[...] = mm(qn)
    tab_ref[:, C:C + G] = mm(kn)
    tab_ref[:, C + G:TW] = jnp.zeros((kn.shape[0], TW - C - G), jnp.float32)


def _topk_body(xb_ref, xa_ref, idx_ref):
    b = pl.program_id(0)
    xb = xb_ref[0]
    xa = xa_ref[0]
    d = lax.dot_general(xb, xa, (((1,), (1,)), ((), ())),
                        preferred_element_type=jnp.float32)
    sqb = jnp.sum(xb * xb, axis=1, keepdims=True)
    sqa = jnp.sum(xa * xa, axis=1)
    nd = 2.0 * d - sqb - sqa[None, :]
    r = nd.shape[0]
    iota = lax.broadcasted_iota(jnp.int32, (r, N), 1)
    cols = []
    for _ in range(K):
        m = jnp.max(nd, axis=1, keepdims=True)
        sel = jnp.where(nd >= m, iota, N)
        a = jnp.min(sel, axis=1, keepdims=True)
        cols.append(a)
        nd = jnp.where(iota == a, _NEG, nd)
    idx = jnp.concatenate(cols, axis=1)
    idx_ref[...] = idx + b * N


def _sc_gather_body(idx_hbm, tab_hbm, out_hbm, idx_v, buf_a, buf_b,
                    sem_a, sem_b):
    wid = lax.axis_index("s") * 2 + lax.axis_index("c")
    wbase = pl.multiple_of(wid * PER_W, CH)
    nch = PER_W // CH

    pltpu.sync_copy(idx_hbm.at[pl.ds(pl.multiple_of(wid * nch, 8), nch)], idx_v)
    pltpu.async_copy(tab_hbm.at[idx_v.at[0]], buf_a, sem_a)

    def body(j, carry):
        i0 = 2 * j
        pltpu.async_copy(tab_hbm.at[idx_v.at[i0 + 1]], buf_b, sem_b)
        pltpu.make_async_copy(tab_hbm.at[idx_v.at[i0]], buf_a, sem_a).wait()
        pltpu.sync_copy(buf_a, out_hbm.at[pl.ds(wbase + i0 * CH, CH)])

        @pl.when(i0 + 2 < nch)
        def _():
            pltpu.async_copy(tab_hbm.at[idx_v.at[i0 + 2]], buf_a, sem_a)

        pltpu.make_async_copy(tab_hbm.at[idx_v.at[i0 + 1]], buf_b, sem_b).wait()
        pltpu.sync_copy(buf_b, out_hbm.at[pl.ds(wbase + (i0 + 1) * CH, CH)])
        return carry

    lax.fori_loop(0, nch // 2, body, 0)


NV = C // 16


def _sc_gatherk_body(idx_hbm, tab_hbm, ppe_hbm, ke1e_hbm, stp_hbm,
                     idx_v, buf_a, buf_b, nodebuf, stbuf, sem_a, sem_b):
    wid = lax.axis_index("s") * 2 + lax.axis_index("c")
    wbase = pl.multiple_of(wid * PER_W, CH)
    nch = PER_W // CH

    pltpu.sync_copy(idx_hbm.at[pl.ds(pl.multiple_of(wid * nch, 8), nch)], idx_v)
    pltpu.async_copy(tab_hbm.at[idx_v.at[0]], buf_a, sem_a)

    def wb(buf, i):
        pltpu.sync_copy(buf.at[:, pl.ds(0, C)],
                        ppe_hbm.at[pl.ds(wbase + i * CH, CH)])
        pltpu.sync_copy(buf.at[:, pl.ds(C, 128)],
                        ke1e_hbm.at[pl.ds(wbase + i * CH, CH)])

    def accum(buf, i, acc):
        node0 = pl.multiple_of((wbase + i * CH) // K, 8)
        pltpu.sync_copy(tab_hbm.at[pl.ds(node0, CH // K)], nodebuf)

        def nbody(nn, acc):
            pv = [nodebuf[nn, pl.ds(16 * v, 16)] for v in range(NV)]

            def ebody(k, acc):
                s, q = acc
                e = nn * K + k
                ns, nq = [], []
                for v in range(NV):
                    d = buf[e, pl.ds(16 * v, 16)] - pv[v]
                    ns.append(s[v] + d)
                    nq.append(q[v] + d * d)
                return (tuple(ns), tuple(nq))

            return lax.fori_loop(0, K, ebody, acc)

        return lax.fori_loop(0, CH // K, nbody, acc)

    zero = jnp.zeros((16,), jnp.float32)
    acc0 = (tuple([zero] * NV), tuple([zero] * NV))

    def body(j, acc):
        i0 = 2 * j
        pltpu.async_copy(tab_hbm.at[idx_v.at[i0 + 1]], buf_b, sem_b)
        pltpu.make_async_copy(tab_hbm.at[idx_v.at[i0]], buf_a, sem_a).wait()
        wb(buf_a, i0)
        acc = accum(buf_a, i0, acc)

        @pl.when(i0 + 2 < nch)
        def _():
            pltpu.async_copy(tab_hbm.at[idx_v.at[i0 + 2]], buf_a, sem_a)

        pltpu.make_async_copy(tab_hbm.at[idx_v.at[i0 + 1]], buf_b, sem_b).wait()
        wb(buf_b, i0 + 1)
        acc = accum(buf_b, i0 + 1, acc)
        return acc

    s, q = lax.fori_loop(0, nch // 2, body, acc0)
    for v in range(NV):
        stbuf[0, pl.ds(16 * v, 16)] = s[v]
        stbuf[1, pl.ds(16 * v, 16)] = q[v]
        for r in range(2, 8):
            stbuf[r, pl.ds(16 * v, 16)] = zero
    pltpu.sync_copy(stbuf, stp_hbm.at[wid])


def _e1_body(ppe_ref, pp_ref, stpb_ref, gbpb_ref, qe1_ref, ke1e_ref, we1_ref,
             u1_ref, st_ref):
    i = pl.program_id(0)
    cnt = jnp.float32(E)
    mean = jnp.sum(stpb_ref[:, 0, :], axis=0)[None, :] / cnt
    ex2 = jnp.sum(stpb_ref[:, 1, :], axis=0)[None, :] / cnt
    inv = lax.rsqrt(ex2 - mean * mean + EPS)
    r = pp_ref.shape[0]
    y = ppe_ref[...].reshape(r, K, C) - pp_ref[...][:, None, :]
    posb = _leaky((y - mean[None, :, :]) * inv[None, :, :]
                  * gbpb_ref[0:1, :][None, :, :] + gbpb_ref[1:2, :][None, :, :])
    u1 = lax.dot_general(posb.reshape(r * K, C), we1_ref[...],
                         (((1,), (1,)), ((), ())),
                         preferred_element_type=jnp.float32).reshape(r, K, G)
    u1 = u1 + ke1e_ref[...].reshape(r, K, 128)[:, :, 0:G] \
        - qe1_ref[...][:, None, :]
    u1_ref[...] = u1
    s = jnp.sum(u1, axis=(0, 1))[None, :]
    s2 = jnp.sum(u1 * u1, axis=(0, 1))[None, :]
    part = jnp.concatenate(
        [jnp.broadcast_to(s, (8, G)), jnp.broadcast_to(s2, (8, G))], axis=0)

    @pl.when(i == 0)
    def _():
        st_ref[...] = jnp.zeros_like(st_ref)

    st_ref[...] += part


def _e2_body(u1_ref, st1_ref, gbe1_ref, we2_ref, u2_ref, st_ref):
    i = pl.program_id(0)
    cnt = jnp.float32(E)
    mean = st1_ref[0:1, :] / cnt
    ex2 = st1_ref[8:9, :] / cnt
    inv = lax.rsqrt(ex2 - mean * mean + EPS)
    u1 = u1_ref[...]
    r = u1.shape[0]
    t1 = _leaky((u1 - mean[None, :, :]) * inv[None, :, :]
                * gbe1_ref[0:1, :][None, :, :] + gbe1_ref[1:2, :][None, :, :])
    u2 = lax.dot_general(t1.reshape(r * K, G), we2_ref[...],
                         (((1,), (1,)), ((), ())),
                         preferred_element_type=jnp.float32).reshape(r, K, G)
    u2_ref[...] = u2
    s = jnp.sum(u2, axis=(0, 1))[None, :]
    s2 = jnp.sum(u2 * u2, axis=(0, 1))[None, :]
    part = jnp.concatenate(
        [jnp.broadcast_to(s, (8, G)), jnp.broadcast_to(s2, (8, G))], axis=0)

    @pl.when(i == 0)
    def _():
        st_ref[...] = jnp.zeros_like(st_ref)

    st_ref[...] += part


def _final_body(u2_ref, st2_ref, gbe2_ref, ppe_ref, pp_ref, stpb_ref,
                gbpb_ref, ve_ref, erep_ref, out_ref):
    cnt = jnp.float32(E)
    mean2 = st2_ref[0:1, :] / cnt
    ex22 = st2_ref[8:9, :] / cnt
    inv2 = lax.rsqrt(ex22 - mean2 * mean2 + EPS)
    u2 = u2_ref[...]
    t2 = _leaky((u2 - mean2[None, :, :]) * inv2[None, :, :]
                * gbe2_ref[0:1, :][None, :, :] + gbe2_ref[1:2, :][None, :, :])
    mx = jnp.max(t2, axis=1, keepdims=True)
    exv = jnp.exp(t2 - mx)
    attn = exv / jnp.sum(exv, axis=1, keepdims=True)
    r = attn.shape[0]
    attn_c = jnp.dot(attn.reshape(r * K, G), erep_ref[...],
                     preferred_element_type=jnp.float32).reshape(r, K, C)
    meanp = jnp.sum(stpb_ref[:, 0, :], axis=0)[None, :] / cnt
    ex2p = jnp.sum(stpb_ref[:, 1, :], axis=0)[None, :] / cnt
    invp = lax.rsqrt(ex2p - meanp * meanp + EPS)
    y = ppe_ref[...].reshape(r, K, C) - pp_ref[...][:, None, :]
    posb = _leaky((y - meanp[None, :, :]) * invp[None, :, :]
                  * gbpb_ref[0:1, :][None, :, :] + gbpb_ref[1:2, :][None, :, :])
    value = ve_ref[...].reshape(r, K, C) + posb
    out_ref[...] = jnp.sum(value * attn_c, axis=1)


def kernel(x, Wq, bq, gq, betaq, Wk, bk, gk, betak, Wv, bv, gv, betav,
           Wpb, bpb, gpb, betapb, We1, be1, ge1, betae1, We2, be2, ge2, betae2):
    f32 = jnp.float32
    xf = x.reshape(BN, C)

    gam = jnp.concatenate([gq, gk, gv, jnp.ones((C,), f32)])
    bet = jnp.concatenate([betaq, betak, betav, jnp.zeros((C,), f32)])
    gb_qkv = jnp.zeros((8, 4 * C), f32).at[0].set(gam).at[1].set(bet)
    gb_pb = jnp.zeros((8, C), f32).at[0].set(gpb).at[1].set(betapb)
    gb_e1 = jnp.zeros((8, G), f32).at[0].set(ge1).at[1].set(betae1)
    gb_e2 = jnp.zeros((8, G), f32).at[0].set(ge2).at[1].set(betae2)
    erep = jnp.repeat(jnp.eye(G, dtype=f32), H, axis=1).reshape(G, C)

    R1 = 512
    wspec = pl.BlockSpec((C, C), lambda i: (0, 0))
    y_all, st_qkv = pl.pallas_call(
        _proj1_body,
        grid=(BN // R1,),
        in_specs=[pl.BlockSpec((R1, C), lambda i: (i, 0)),
                  wspec, wspec, wspec, wspec],
        out_specs=[pl.BlockSpec((R1, 4 * C), lambda i: (i, 0)),
                   pl.BlockSpec((16, 4 * C), lambda i: (0, 0))],
        out_shape=[jax.ShapeDtypeStruct((BN, 4 * C), f32),
                   jax.ShapeDtypeStruct((16, 4 * C), f32)],
    )(xf, Wq, Wk, Wv, Wpb)

    R2 = 512
    kn, vn, tab, qe1 = pl.pallas_call(
        _proj2_body,
        grid=(BN // R2,),
        in_specs=[pl.BlockSpec((R2, 4 * C), lambda i: (i, 0)),
                  pl.BlockSpec((16, 4 * C), lambda i: (0, 0)),
                  pl.BlockSpec((8, 4 * C), lambda i: (0, 0)),
                  pl.BlockSpec((G, C), lambda i: (0, 0))],
        out_specs=[pl.BlockSpec((R2, C), lambda i: (i, 0)),
                   pl.BlockSpec((R2, C), lambda i: (i, 0)),
                   pl.BlockSpec((R2, TW), lambda i: (i, 0)),
                   pl.BlockSpec((R2, G), lambda i: (i, 0))],
        out_shape=[jax.ShapeDtypeStruct((BN, C), f32),
                   jax.ShapeDtypeStruct((BN, C), f32),
                   jax.ShapeDtypeStruct((BN, TW), f32),
                   jax.ShapeDtypeStruct((BN, G), f32)],
    )(y_all, st_qkv, gb_qkv, We1)

    R3 = 256

    def _run_topk(feat):
        return pl.pallas_call(
            _topk_body,
            grid=(B, N // R3),
            in_specs=[pl.BlockSpec((1, R3, C), lambda b, j: (b, j, 0)),
                      pl.BlockSpec((1, N, C), lambda b, j: (b, 0, 0))],
            out_specs=pl.BlockSpec((R3, K), lambda b, j: (b * (N // R3) + j, 0)),
            out_shape=jax.ShapeDtypeStruct((BN, K), jnp.int32),
        )(feat, feat)

    idxg = _run_topk(kn.reshape(B, N, C))
    idxgv = _run_topk(vn.reshape(B, N, C))

    mesh = plsc.VectorSubcoreMesh(core_axis_name="c", subcore_axis_name="s")

    def _scratch(width):
        return [pltpu.VMEM((E // NW // CH, CH), jnp.int32),
                pltpu.VMEM((CH, width), f32),
                pltpu.VMEM((CH, width), f32),
                pltpu.SemaphoreType.DMA,
                pltpu.SemaphoreType.DMA]

    gather_k = functools.partial(
        pl.kernel, mesh=mesh,
        out_type=[jax.ShapeDtypeStruct((E, C), f32),
                  jax.ShapeDtypeStruct((E, 128), f32),
                  jax.ShapeDtypeStruct((NW, 8, C), f32)],
        scratch_types=[pltpu.VMEM((E // NW // CH, CH), jnp.int32),
                       pltpu.VMEM((CH, TW), f32),
                       pltpu.VMEM((CH, TW), f32),
                       pltpu.VMEM((CH // K, TW), f32),
                       pltpu.VMEM((8, C), f32),
                       pltpu.SemaphoreType.DMA,
                       pltpu.SemaphoreType.DMA],
    )(_sc_gatherk_body)
    ppe_f, ke1e_f, stp = gather_k(idxg.reshape(E // CH, CH), tab)

    gather_v = functools.partial(
        pl.kernel, mesh=mesh,
        out_type=jax.ShapeDtypeStruct((E, C), f32),
        scratch_types=_scratch(C),
    )(_sc_gather_body)
    ve_f = gather_v(idxgv.reshape(E // CH, CH), vn)

    R4 = 256
    u1e, st_u1 = pl.pallas_call(
        _e1_body,
        grid=(BN // R4,),
        in_specs=[pl.BlockSpec((R4 * K, C), lambda i: (i, 0)),
                  pl.BlockSpec((R4, C), lambda i: (i, 0)),
                  pl.BlockSpec((NW, 8, C), lambda i: (0, 0, 0)),
                  pl.BlockSpec((8, C), lambda i: (0, 0)),
                  pl.BlockSpec((R4, G), lambda i: (i, 0)),
                  pl.BlockSpec((R4 * K, 128), lambda i: (i, 0)),
                  pl.BlockSpec((G, C), lambda i: (0, 0))],
        out_specs=[pl.BlockSpec((R4, K, G), lambda i: (i, 0, 0)),
                   pl.BlockSpec((16, G), lambda i: (0, 0))],
        out_shape=[jax.ShapeDtypeStruct((BN, K, G), f32),
                   jax.ShapeDtypeStruct((16, G), f32)],
    )(ppe_f, tab, stp, gb_pb, qe1, ke1e_f, We1)

    R5 = 512
    u2e, st_u2 = pl.pallas_call(
        _e2_body,
        grid=(BN // R5,),
        in_specs=[pl.BlockSpec((R5, K, G), lambda i: (i, 0, 0)),
                  pl.BlockSpec((16, G), lambda i: (0, 0)),
                  pl.BlockSpec((8, G), lambda i: (0, 0)),
                  pl.BlockSpec((G, G), lambda i: (0, 0))],
        out_specs=[pl.BlockSpec((R5, K, G), lambda i: (i, 0, 0)),
                   pl.BlockSpec((16, G), lambda i: (0, 0))],
        out_shape=[jax.ShapeDtypeStruct((BN, K, G), f32),
                   jax.ShapeDtypeStruct((16, G), f32)],
    )(u1e, st_u1, gb_e1, We2)

    out = pl.pallas_call(
        _final_body,
        grid=(BN // R4,),
        in_specs=[pl.BlockSpec((R4, K, G), lambda i: (i, 0, 0)),
                  pl.BlockSpec((16, G), lambda i: (0, 0)),
                  pl.BlockSpec((8, G), lambda i: (0, 0)),
                  pl.BlockSpec((R4 * K, C), lambda i: (i, 0)),
                  pl.BlockSpec((R4, C), lambda i: (i, 0)),
                  pl.BlockSpec((NW, 8, C), lambda i: (0, 0, 0)),
                  pl.BlockSpec((8, C), lambda i: (0, 0)),
                  pl.BlockSpec((R4 * K, C), lambda i: (i, 0)),
                  pl.BlockSpec((G, C), lambda i: (0, 0))],
        out_specs=pl.BlockSpec((R4, C), lambda i: (i, 0)),
        out_shape=jax.ShapeDtypeStruct((BN, C), f32),
    )(u2e, st_u2, gb_e2, ppe_f, tab, stp, gb_pb, ve_f, erep)

    return out.reshape(B, N, C)
